# baseline (speedup 1.0000x reference)
import functools

import jax
import jax.numpy as jnp
from jax import lax
from jax.experimental import pallas as pl
from jax.experimental.pallas import tpu as pltpu
from jax.experimental.pallas import tpu_sc as plsc

_VOTER_INPUT = 8
_SIGNAL_COUNT = 64
_N = 819200
_ROWS = 256

_NC = 2
_NS = 16
_NW = _NC * _NS

_CHUNK = 512
_IDXW = 128
_NGRP = _CHUNK // _IDXW
_B_PER_W = _N // _NW
_NCHUNK = _B_PER_W // _CHUNK
_NPAIR = _NCHUNK // 2


def _emb_body(
    x_hbm, table_hbm, out_hbm, xv, id0, id1, rv0, rv1, sg, so0, so1
):
    wid = lax.axis_index("s") * _NC + lax.axis_index("c")
    wbase = wid * _B_PER_W

    lane = lax.iota(jnp.int32, 16)
    ids = (id0, id1)
    rvs = (rv0, rv1)
    sos = (so0, so1)

    def pack_chunk(c, b):
        row0 = pl.multiple_of(wbase + c * _CHUNK, _CHUNK)
        pltpu.sync_copy(x_hbm.at[pl.ds(row0, _CHUNK)], xv)
        tbase = wid * _ROWS
        for k in range(_CHUNK // 16):
            acc = jnp.full((16,), tbase, jnp.int32)
            for j in range(_VOTER_INPUT):
                col = plsc.load_gather(
                    xv, [lane + (k * 16), jnp.full((16,), j, jnp.int32)]
                )
                acc = acc + col * (1 << (_VOTER_INPUT - 1 - j))
            g, o = divmod(k * 16, _IDXW)
            ids[b][g, pl.ds(o, 16)] = acc

    def fire_gathers(c, b):
        return [
            pltpu.async_copy(
                table_hbm.at[ids[b].at[g]],
                rvs[b].at[pl.ds(g * _IDXW, _IDXW), 0],
                sg,
            )
            for g in range(_NGRP)
        ]

    def issue_out(c, b):
        row0 = pl.multiple_of(wbase + c * _CHUNK, _CHUNK)
        pltpu.async_copy(rvs[b], out_hbm.at[pl.ds(row0, _CHUNK)], sos[b])

    def drain_out(b):
        pltpu.make_async_copy(
            rvs[b], out_hbm.at[pl.ds(0, _CHUNK)], sos[b]
        ).wait()

    pack_chunk(0, 0)
    for d in fire_gathers(0, 0):
        d.wait()
    pack_chunk(1, 1)
    descs = fire_gathers(1, 1)
    issue_out(0, 0)
    for d in descs:
        d.wait()

    def pair(p, carry):
        for b in range(2):
            c = 2 * p + b
            pack_chunk(c, b)
            drain_out(b)
            descs = fire_gathers(c, b)
            issue_out(c - 1, 1 - b)
            for d in descs:
                d.wait()
        return carry

    lax.fori_loop(1, _NPAIR, pair, 0)

    pltpu.make_async_copy(rv0, out_hbm.at[pl.ds(0, _CHUNK)], so0).wait()
    rlast = pl.multiple_of(wbase + (_NCHUNK - 1) * _CHUNK, _CHUNK)
    pltpu.sync_copy(rv1, out_hbm.at[pl.ds(rlast, _CHUNK)])


@jax.jit
def _emb_lookup(x, table_rep):
    mesh = plsc.VectorSubcoreMesh(core_axis_name="c", subcore_axis_name="s")
    run = functools.partial(
        pl.kernel,
        mesh=mesh,
        out_type=jax.ShapeDtypeStruct((_N, 1, _SIGNAL_COUNT), jnp.float32),
        scratch_types=[
            pltpu.VMEM((_CHUNK, _VOTER_INPUT), jnp.int32),
            pltpu.VMEM((_NGRP, _IDXW), jnp.int32),
            pltpu.VMEM((_NGRP, _IDXW), jnp.int32),
            pltpu.VMEM((_CHUNK, 1, _SIGNAL_COUNT), jnp.float32),
            pltpu.VMEM((_CHUNK, 1, _SIGNAL_COUNT), jnp.float32),
            pltpu.SemaphoreType.DMA,
            pltpu.SemaphoreType.DMA,
            pltpu.SemaphoreType.DMA,
        ],
        compiler_params=pltpu.CompilerParams(
            needs_layout_passes=False, use_tc_tiling_on_sc=False
        ),
    )(_emb_body)
    return run(x, table_rep)


def kernel(x, emb_weight):
    x = x.reshape(_N, _VOTER_INPUT).astype(jnp.int32)
    table_rep = jnp.tile(emb_weight, (_NW, 1))
    return _emb_lookup(x, table_rep)


# speedup vs baseline: 1.7914x; 1.7914x over previous
import functools

import jax
import jax.numpy as jnp
from jax import lax
from jax.experimental import pallas as pl
from jax.experimental.pallas import tpu as pltpu
from jax.experimental.pallas import tpu_sc as plsc

_VOTER_INPUT = 8
_SIGNAL_COUNT = 64
_N = 819200
_ROWS = 256

_NC = 2
_NS = 16
_NW = _NC * _NS

_CHUNK = 512
_IDXW = 128
_NGRP = _CHUNK // _IDXW
_B_PER_W = _N // _NW
_NCHUNK = _B_PER_W // _CHUNK
_NPAIR = _NCHUNK // 2


def _emb_body(
    x_hbm, table_hbm, out_hbm, xv, id0, id1, rv0, rv1, sg, so0, so1
):
    wid = lax.axis_index("s") * _NC + lax.axis_index("c")
    wbase = wid * _B_PER_W

    lane = lax.iota(jnp.int32, 16)
    ids = (id0, id1)
    rvs = (rv0, rv1)
    sos = (so0, so1)

    def pack_chunk(c, b):
        row0 = pl.multiple_of(wbase + c * _CHUNK, _CHUNK)
        pltpu.sync_copy(x_hbm.at[pl.ds(row0, _CHUNK)], xv)
        tbase = wid * _ROWS
        for k in range(_CHUNK // 16):
            acc = jnp.full((16,), tbase, jnp.int32)
            for j in range(_VOTER_INPUT):
                col = plsc.load_gather(
                    xv, [lane + (k * 16), jnp.full((16,), j, jnp.int32)]
                )
                acc = acc + col * (1 << (_VOTER_INPUT - 1 - j))
            g, o = divmod(k * 16, _IDXW)
            ids[b][g, pl.ds(o, 16)] = acc

    def fire_gathers(c, b):
        return [
            pltpu.async_copy(
                table_hbm.at[ids[b].at[g]],
                rvs[b].at[pl.ds(g * _IDXW, _IDXW)],
                sg,
            )
            for g in range(_NGRP)
        ]

    def issue_out(c, b):
        row0 = pl.multiple_of(wbase + c * _CHUNK, _CHUNK)
        pltpu.async_copy(rvs[b], out_hbm.at[pl.ds(row0, _CHUNK)], sos[b])

    def drain_out(b):
        pltpu.make_async_copy(
            rvs[b], out_hbm.at[pl.ds(0, _CHUNK)], sos[b]
        ).wait()

    pack_chunk(0, 0)
    for d in fire_gathers(0, 0):
        d.wait()
    pack_chunk(1, 1)
    descs = fire_gathers(1, 1)
    issue_out(0, 0)
    for d in descs:
        d.wait()

    def pair(p, carry):
        for b in range(2):
            c = 2 * p + b
            pack_chunk(c, b)
            drain_out(b)
            descs = fire_gathers(c, b)
            issue_out(c - 1, 1 - b)
            for d in descs:
                d.wait()
        return carry

    lax.fori_loop(1, _NPAIR, pair, 0)

    pltpu.make_async_copy(rv0, out_hbm.at[pl.ds(0, _CHUNK)], so0).wait()
    rlast = pl.multiple_of(wbase + (_NCHUNK - 1) * _CHUNK, _CHUNK)
    pltpu.sync_copy(rv1, out_hbm.at[pl.ds(rlast, _CHUNK)])


@jax.jit
def _emb_lookup(x, table_rep):
    mesh = plsc.VectorSubcoreMesh(core_axis_name="c", subcore_axis_name="s")
    run = functools.partial(
        pl.kernel,
        mesh=mesh,
        out_type=jax.ShapeDtypeStruct((_N, _SIGNAL_COUNT), jnp.float32),
        scratch_types=[
            pltpu.VMEM((_CHUNK, _VOTER_INPUT), jnp.int32),
            pltpu.VMEM((_NGRP, _IDXW), jnp.int32),
            pltpu.VMEM((_NGRP, _IDXW), jnp.int32),
            pltpu.VMEM((_CHUNK, _SIGNAL_COUNT), jnp.float32),
            pltpu.VMEM((_CHUNK, _SIGNAL_COUNT), jnp.float32),
            pltpu.SemaphoreType.DMA,
            pltpu.SemaphoreType.DMA,
            pltpu.SemaphoreType.DMA,
        ],
        compiler_params=pltpu.CompilerParams(
            needs_layout_passes=False, use_tc_tiling_on_sc=False
        ),
    )(_emb_body)
    return run(x, table_rep)


def kernel(x, emb_weight):
    x = x.reshape(_N, _VOTER_INPUT).astype(jnp.int32)
    table_rep = jnp.tile(emb_weight, (_NW, 1))
    out = _emb_lookup(x, table_rep)
    return out.reshape(_N, 1, _SIGNAL_COUNT)
